# Initial kernel scaffold; baseline (speedup 1.0000x reference)
import jax
import jax.numpy as jnp
from jax.experimental import pallas as pl


def kernel(x_feats, ln1_w, ln1_b, W1, b1, W2, b2, Wsub, bsub, coords, sel_idx):
    raise NotImplementedError("write your pallas kernel here")


# speedup vs baseline: 9.2690x; 9.2690x over previous
import functools

import jax
import jax.numpy as jnp
import numpy as np
from jax import lax
from jax.experimental import pallas as pl
from jax.experimental.pallas import tpu as pltpu
from jax.experimental.pallas import tpu_sc as plsc

GRID = 64
PG1 = GRID + 2
C = 16
COUT1 = 128
T1 = PG1 ** 3

NC, NS, L = 2, 16, 16
NW = NC * NS

DELTAS1 = [(dx * PG1 + dy) * PG1 + dz
           for dx in (-1, 0, 1) for dy in (-1, 0, 1) for dz in (-1, 0, 1)]

_mesh = plsc.VectorSubcoreMesh(core_axis_name="c", subcore_axis_name="s")
_SC_PARAMS = pltpu.CompilerParams(needs_layout_passes=False)


def _wid():
    return lax.axis_index("s") * NC + lax.axis_index("c")


def _ceil_to(x, m):
    return (x + m - 1) // m * m


def _ka_body(x_ref, cd_ref, lnw_ref, lnb_ref, ws_ref, bs_ref,
             sd_ref, h1_ref, pk_ref):
    x = x_ref[...]
    sd_ref[...] = jnp.dot(x, ws_ref[...], preferred_element_type=jnp.float32) + bs_ref[...]
    m = jnp.mean(x, axis=-1, keepdims=True)
    v = jnp.mean((x - m) ** 2, axis=-1, keepdims=True)
    h = (x - m) * lax.rsqrt(v + 1e-6) * lnw_ref[...] + lnb_ref[...]
    h1_ref[...] = h * jax.nn.sigmoid(h)
    cd = cd_ref[...]
    xs, ys, zs = cd[:, 1:2], cd[:, 2:3], cd[:, 3:4]
    pk = ((xs + 1) * PG1 + (ys + 1)) * PG1 + (zs + 1)
    pk_ref[...] = jnp.broadcast_to(pk, pk.shape[:1] + (8,))


def _run_ka(x_feats, coords, ln1_w, ln1_b, Wsub, bsub):
    N = x_feats.shape[0]
    BA = 2048
    full = lambda s: pl.BlockSpec(s, lambda i: (0,) * len(s))
    row = lambda w: pl.BlockSpec((BA, w), lambda i: (i, 0))
    return pl.pallas_call(
        _ka_body,
        grid=(N // BA,),
        in_specs=[row(16), row(4), full((16,)), full((16,)), full((16, 8)),
                  full((8,))],
        out_specs=[row(8), row(16), row(8)],
        out_shape=[jax.ShapeDtypeStruct((N, 8), jnp.float32),
                   jax.ShapeDtypeStruct((N, 16), jnp.float32),
                   jax.ShapeDtypeStruct((N, 8), jnp.int32)],
    )(x_feats, coords, ln1_w, ln1_b, Wsub, bsub)


def _make_map_builder(npts, nvalid, cpt, pad_val, name):
    CH = 2048
    npts_pad = _ceil_to(npts, CH)

    def body(keys_hbm, map_hbm, mslice, kbuf, sem):
        wid = _wid()
        base_cell = wid * cpt
        padv = jnp.full((L,), pad_val, jnp.int32)

        def memset(i, _):
            mslice[pl.ds(i * L, L)] = padv
            return 0
        lax.fori_loop(0, cpt // L, memset, 0)

        iota = lax.iota(jnp.int32, L)

        def chunk(ci, _):
            pltpu.async_copy(keys_hbm.at[pl.ds(ci * CH, CH)], kbuf, sem).wait()

            def vec(j, _):
                kv = kbuf[pl.ds(j * L, L)]
                gidx = ci * CH + j * L + iota
                local = kv - base_cell
                mask = (local >= 0) & (local < cpt) & (gidx < nvalid)
                local = jnp.where(mask, local, 0)
                plsc.store_scatter(mslice, [local], gidx, mask=mask)
                return 0
            lax.fori_loop(0, CH // L, vec, 0)
            return 0
        lax.fori_loop(0, npts_pad // CH, chunk, 0)
        pltpu.async_copy(mslice, map_hbm.at[pl.ds(base_cell, cpt)], sem).wait()

    return functools.partial(
        pl.kernel, body, mesh=_mesh, name=name, compiler_params=_SC_PARAMS,
        out_type=jax.ShapeDtypeStruct((cpt * NW,), jnp.int32),
        scratch_types=[pltpu.VMEM((cpt,), jnp.int32),
                       pltpu.VMEM((CH,), jnp.int32),
                       pltpu.SemaphoreType.DMA])()


def _make_g1(npts, name):
    CH = 128
    per_tile = npts // NW
    nchunks = per_tile // CH
    nk = 27
    KW = 512

    def body(keys_hbm, map_hbm, tbl_hbm, g_hbm,
             kbuf, nkbuf, idxbuf, rb, g512, sem, semw):
        wid = _wid()
        base = wid * per_tile
        zv = jnp.zeros((L,), jnp.float32)

        def zrow(i, _):
            for t in range((KW - nk * C) // L):
                g512[i, pl.ds(nk * C + t * L, L)] = zv
            return 0
        lax.fori_loop(0, CH, zrow, 0)

        def relayout(k, buf):
            def rowcopy(r, _):
                g512[r, pl.ds(k * C, C)] = rb[buf, r, pl.ds(0, C)]
                return 0
            lax.fori_loop(0, CH, rowcopy, 0)

        def chunk(ci, wcnt):
            r0 = base + ci * CH
            pltpu.async_copy(keys_hbm.at[pl.ds(r0, CH)], kbuf, sem).wait()

            def vec(j, _):
                kv = kbuf[pl.ds(j * L, L)]
                for k in range(nk):
                    nkbuf[k, pl.ds(j * L, L)] = kv + DELTAS1[k]
                return 0
            lax.fori_loop(0, CH // L, vec, 0)

            d1 = [pltpu.async_copy(map_hbm.at[nkbuf.at[k]], idxbuf.at[k], sem)
                  for k in range(nk)]
            for d in d1:
                d.wait()
            descs = [None, None]
            descs[0] = pltpu.async_copy(tbl_hbm.at[idxbuf.at[0]], rb.at[0], sem)
            for k in range(nk):
                descs[k % 2].wait()
                if k + 1 < nk:
                    descs[(k + 1) % 2] = pltpu.async_copy(
                        tbl_hbm.at[idxbuf.at[k + 1]], rb.at[(k + 1) % 2], sem)
                relayout(k, k % 2)

            @pl.when(wcnt > 0)
            def _():
                pltpu.make_async_copy(g512, g_hbm.at[pl.ds(0, CH)], semw).wait()
            pltpu.async_copy(g512, g_hbm.at[pl.ds(r0, CH)], semw)
            return wcnt + 1
        nw = lax.fori_loop(0, nchunks, chunk, 0)

        @pl.when(nw > 0)
        def _():
            pltpu.make_async_copy(g512, g_hbm.at[pl.ds(0, CH)], semw).wait()

    return functools.partial(
        pl.kernel, body, mesh=_mesh, name=name, compiler_params=_SC_PARAMS,
        out_type=jax.ShapeDtypeStruct((npts, KW), jnp.float32),
        scratch_types=[pltpu.VMEM((CH,), jnp.int32),
                       pltpu.VMEM((nk, CH), jnp.int32),
                       pltpu.VMEM((nk, CH), jnp.int32),
                       pltpu.VMEM((2, CH, 128), jnp.float32),
                       pltpu.VMEM((CH, KW), jnp.float32),
                       pltpu.SemaphoreType.DMA,
                       pltpu.SemaphoreType.DMA])()


def _make_g2(npts, name):
    CH = 32
    per_tile = npts // NW
    nchunks = per_tile // CH
    nk = 27
    KW = nk * 128

    def body(keys_hbm, map_hbm, tbl_hbm, g_hbm,
             kbuf, nkbuf, idxbuf, gbig, sem, semw):
        wid = _wid()
        base = wid * per_tile

        def chunk(ci, wcnt):
            r0 = base + ci * CH
            pltpu.async_copy(keys_hbm.at[pl.ds(r0, CH)], kbuf, sem).wait()

            def vec(j, _):
                kv = kbuf[pl.ds(j * L, L)]
                for k in range(nk):
                    nkbuf[k, pl.ds(j * L, L)] = kv + DELTAS1[k]
                return 0
            lax.fori_loop(0, CH // L, vec, 0)

            d1 = [pltpu.async_copy(map_hbm.at[nkbuf.at[k]], idxbuf.at[k], sem)
                  for k in range(nk)]
            for d in d1:
                d.wait()

            @pl.when(wcnt > 0)
            def _():
                pltpu.make_async_copy(gbig, g_hbm.at[pl.ds(0, CH)], semw).wait()

            d2 = [pltpu.async_copy(
                tbl_hbm.at[idxbuf.at[k]],
                gbig.at[pl.ds(0, CH), pl.ds(k * 128, 128)], sem)
                  for k in range(nk)]
            for d in d2:
                d.wait()
            pltpu.async_copy(gbig, g_hbm.at[pl.ds(r0, CH)], semw)
            return wcnt + 1
        nw = lax.fori_loop(0, nchunks, chunk, 0)

        @pl.when(nw > 0)
        def _():
            pltpu.make_async_copy(gbig, g_hbm.at[pl.ds(0, CH)], semw).wait()

    return functools.partial(
        pl.kernel, body, mesh=_mesh, name=name, compiler_params=_SC_PARAMS,
        out_type=jax.ShapeDtypeStruct((npts, KW), jnp.float32),
        scratch_types=[pltpu.VMEM((CH,), jnp.int32),
                       pltpu.VMEM((nk, CH), jnp.int32),
                       pltpu.VMEM((nk, CH), jnp.int32),
                       pltpu.VMEM((CH, KW), jnp.float32),
                       pltpu.SemaphoreType.DMA,
                       pltpu.SemaphoreType.DMA])()


def _make_extract(mpad, name):
    CH = 128
    per_tile = mpad // NW
    nchunks = per_tile // CH

    def body(sel_hbm, oc_hbm, out_hbm, sbuf, pbuf, wb, ob, sem):
        wid = _wid()
        base = wid * per_tile
        iota = lax.iota(jnp.int32, L)

        def chunk(ci, _):
            r0 = base + ci * CH
            pltpu.async_copy(sel_hbm.at[pl.ds(r0, CH)], sbuf, sem).wait()

            def vec(j, _):
                sv = sbuf[pl.ds(j * L, L)]
                pbuf[pl.ds(j * L, L)] = lax.shift_right_logical(sv, 3)
                return 0
            lax.fori_loop(0, CH // L, vec, 0)
            pltpu.async_copy(oc_hbm.at[pbuf], wb, sem).wait()

            def vec2(j, _):
                sv = sbuf[pl.ds(j * L, L)]
                rows = j * L + iota
                cbase = (sv & 7) * C
                for c in range(C):
                    v = plsc.load_gather(wb, [rows, cbase + c])
                    plsc.store_scatter(ob, [rows, jnp.full((L,), c, jnp.int32)], v)
                return 0
            lax.fori_loop(0, CH // L, vec2, 0)
            pltpu.async_copy(ob, out_hbm.at[pl.ds(r0, CH)], sem).wait()
            return 0
        lax.fori_loop(0, nchunks, chunk, 0)

    return functools.partial(
        pl.kernel, body, mesh=_mesh, name=name, compiler_params=_SC_PARAMS,
        out_type=jax.ShapeDtypeStruct((mpad, C), jnp.float32),
        scratch_types=[pltpu.VMEM((CH,), jnp.int32),
                       pltpu.VMEM((CH,), jnp.int32),
                       pltpu.VMEM((CH, 128), jnp.float32),
                       pltpu.VMEM((CH, C), jnp.float32),
                       pltpu.SemaphoreType.DMA])()


def _kc_body(g_ref, w_ref, b_ref, o_ref):
    o_ref[...] = jnp.dot(g_ref[...], w_ref[...],
                         preferred_element_type=jnp.float32) + b_ref[...]


def _run_matmul(G, Wr, b, BT, name):
    n, kdim = G.shape
    cout = Wr.shape[1]
    full = lambda s: pl.BlockSpec(s, lambda i: (0,) * len(s))
    return pl.pallas_call(
        _kc_body,
        grid=(n // BT,),
        in_specs=[pl.BlockSpec((BT, kdim), lambda i: (i, 0)),
                  full((kdim, cout)), full((cout,))],
        out_specs=pl.BlockSpec((BT, cout), lambda i: (i, 0)),
        out_shape=jax.ShapeDtypeStruct((n, cout), jnp.float32),
        name=name,
    )(G, Wr, b)


def _kd_body(o1_ref, msk_ref, f_ref):
    o = o1_ref[...]
    msk = msk_ref[...]
    blocks = []
    for j in range(8):
        b = o[:, 16 * j:16 * j + 16]
        m = jnp.mean(b, axis=-1, keepdims=True)
        v = jnp.mean((b - m) ** 2, axis=-1, keepdims=True)
        h = (b - m) * lax.rsqrt(v + 1e-6)
        h = h * jax.nn.sigmoid(h)
        keep = (msk[:, j:j + 1] >= 0).astype(jnp.float32)
        blocks.append(h * keep)
    f_ref[...] = jnp.concatenate(blocks, axis=1)


def _run_kd(out1, selmask):
    n = out1.shape[0]
    BT = 1024
    return pl.pallas_call(
        _kd_body,
        grid=(n // BT,),
        in_specs=[pl.BlockSpec((BT, 128), lambda i: (i, 0)),
                  pl.BlockSpec((BT, 8), lambda i: (i, 0))],
        out_specs=pl.BlockSpec((BT, 128), lambda i: (i, 0)),
        out_shape=jax.ShapeDtypeStruct((n, 128), jnp.float32),
        name="ln2",
    )(out1, selmask)


def _ke_body(g_ref, w_ref, b_ref, x_ref, o_ref):
    o = jnp.dot(g_ref[...], w_ref[...], preferred_element_type=jnp.float32) + b_ref[...]
    x = x_ref[...]
    cols = []
    for a in range(8):
        cols.append(jnp.broadcast_to(x[:, 2 * a:2 * a + 1], (x.shape[0], 8)))
        cols.append(jnp.broadcast_to(x[:, 2 * a + 1:2 * a + 2], (x.shape[0], 8)))
    o_ref[...] = o + jnp.concatenate(cols, axis=1)


def _run_ke(G2, Vall, b2t, x_feats):
    n = G2.shape[0]
    BT = 512
    full = lambda s: pl.BlockSpec(s, lambda i: (0,) * len(s))
    return pl.pallas_call(
        _ke_body,
        grid=(n // BT,),
        in_specs=[pl.BlockSpec((BT, 27 * 128), lambda i: (i, 0)),
                  full((27 * 128, 128)), full((128,)),
                  pl.BlockSpec((BT, 16), lambda i: (i, 0))],
        out_specs=pl.BlockSpec((BT, 128), lambda i: (i, 0)),
        out_shape=jax.ShapeDtypeStruct((n, 128), jnp.float32),
        name="conv2mm",
    )(G2, Vall, b2t, x_feats)


CPT1 = _ceil_to(_ceil_to(T1, NW) // NW, 128)

_KIDX = np.full((27, 8, 8), 27, np.int64)
for _ei, (_ex, _ey, _ez) in enumerate(
        [(x, y, z) for x in (-1, 0, 1) for y in (-1, 0, 1) for z in (-1, 0, 1)]):
    for _b in range(8):
        _bv = ((_b >> 2) & 1, (_b >> 1) & 1, _b & 1)
        for _a in range(8):
            _av = ((_a >> 2) & 1, (_a >> 1) & 1, _a & 1)
            _d = (_bv[0] - _av[0] + 2 * _ex, _bv[1] - _av[1] + 2 * _ey,
                  _bv[2] - _av[2] + 2 * _ez)
            if all(-1 <= t <= 1 for t in _d):
                _KIDX[_ei, _b, _a] = (_d[0] + 1) * 9 + (_d[1] + 1) * 3 + (_d[2] + 1)


def kernel(x_feats, ln1_w, ln1_b, W1, b1, W2, b2, Wsub, bsub, coords, sel_idx):
    N = x_feats.shape[0]
    M = sel_idx.shape[0]
    Mpad = _ceil_to(M, NW * 128)

    subdiv, h1, pk8 = _run_ka(x_feats, coords.astype(jnp.int32), ln1_w, ln1_b,
                              Wsub, bsub)
    pkey = pk8[:, 0]

    sel = sel_idx.astype(jnp.int32)
    selpad = jnp.pad(sel, (0, Mpad - M))
    map1 = _make_map_builder(N, N, CPT1, N, "map1")(pkey)
    selmask = _make_map_builder(Mpad, M, (8 * N) // NW, -1, "selmask")(selpad)

    h1w = jnp.pad(h1, ((0, 8), (0, 128 - C)))
    G1 = _make_g1(N, "im2col1")(pkey, map1, h1w)
    W1r = jnp.pad(W1.reshape(27 * C, COUT1), ((0, 512 - 27 * C), (0, 0)))
    out1 = _run_matmul(G1, W1r, b1, 1024, "conv1mm")

    F = _run_kd(out1, selmask.reshape(N, 8))
    Fp = jnp.pad(F, ((0, 8), (0, 0)))
    G2 = _make_g2(N, "im2col2")(pkey, map1, Fp)

    W2z = jnp.concatenate([W2, jnp.zeros((1, C, C), W2.dtype)], axis=0)
    Vall = W2z[jnp.asarray(_KIDX)]
    Vall = Vall.transpose(0, 1, 3, 2, 4).reshape(27 * 128, 128)
    b2t = jnp.tile(b2, 8)

    out_ch = _run_ke(G2, Vall, b2t, x_feats)
    out_chp = jnp.pad(out_ch, ((0, 8), (0, 0)))
    out = _make_extract(Mpad, "extract")(selpad, out_chp)
    return out[:M], subdiv


# speedup vs baseline: 189.4724x; 20.4414x over previous
import functools

import jax
import jax.numpy as jnp
import numpy as np
from jax import lax
from jax.experimental import pallas as pl
from jax.experimental.pallas import tpu as pltpu
from jax.experimental.pallas import tpu_sc as plsc

GRID = 64
PG1 = GRID + 2
C = 16
COUT1 = 128
T1 = PG1 ** 3

NC, NS, L = 2, 16, 16
NW = NC * NS

DELTAS1 = [(dx * PG1 + dy) * PG1 + dz
           for dx in (-1, 0, 1) for dy in (-1, 0, 1) for dz in (-1, 0, 1)]

_mesh = plsc.VectorSubcoreMesh(core_axis_name="c", subcore_axis_name="s")
_SC_PARAMS = pltpu.CompilerParams(needs_layout_passes=False)


def _wid():
    return lax.axis_index("s") * NC + lax.axis_index("c")


def _ceil_to(x, m):
    return (x + m - 1) // m * m


def _ka_body(x_ref, cd_ref, lnw_ref, lnb_ref, ws_ref, bs_ref,
             sd_ref, h1_ref, pk_ref):
    x = x_ref[...]
    sd_ref[...] = jnp.dot(x, ws_ref[...], preferred_element_type=jnp.float32) + bs_ref[...]
    m = jnp.mean(x, axis=-1, keepdims=True)
    v = jnp.mean((x - m) ** 2, axis=-1, keepdims=True)
    h = (x - m) * lax.rsqrt(v + 1e-6) * lnw_ref[...] + lnb_ref[...]
    h1_ref[...] = h * jax.nn.sigmoid(h)
    cd = cd_ref[...]
    xs, ys, zs = cd[:, 1:2], cd[:, 2:3], cd[:, 3:4]
    pk = ((xs + 1) * PG1 + (ys + 1)) * PG1 + (zs + 1)
    pk_ref[...] = jnp.broadcast_to(pk, pk.shape[:1] + (8,))


def _run_ka(x_feats, coords, ln1_w, ln1_b, Wsub, bsub):
    N = x_feats.shape[0]
    BA = 2048
    full = lambda s: pl.BlockSpec(s, lambda i: (0,) * len(s))
    row = lambda w: pl.BlockSpec((BA, w), lambda i: (i, 0))
    return pl.pallas_call(
        _ka_body,
        grid=(N // BA,),
        in_specs=[row(16), row(4), full((16,)), full((16,)), full((16, 8)),
                  full((8,))],
        out_specs=[row(8), row(16), row(8)],
        out_shape=[jax.ShapeDtypeStruct((N, 8), jnp.float32),
                   jax.ShapeDtypeStruct((N, 16), jnp.float32),
                   jax.ShapeDtypeStruct((N, 8), jnp.int32)],
    )(x_feats, coords, ln1_w, ln1_b, Wsub, bsub)


def _make_map_builder(npts, nvalid, cpt, pad_val, name, pad_spread=0):
    CH = 2048
    npts_pad = _ceil_to(npts, CH)

    def body(keys_hbm, map_hbm, mslice, kbuf, sem):
        wid = _wid()
        base_cell = wid * cpt
        iota0 = lax.iota(jnp.int32, L)

        def memset(i, _):
            padv = jnp.full((L,), pad_val, jnp.int32)
            if pad_spread:
                padv = padv + ((i * L + iota0) & (pad_spread - 1))
            mslice[pl.ds(i * L, L)] = padv
            return 0
        lax.fori_loop(0, cpt // L, memset, 0)

        iota = iota0

        def chunk(ci, _):
            pltpu.async_copy(keys_hbm.at[pl.ds(ci * CH, CH)], kbuf, sem).wait()

            def vec(j, _):
                kv = kbuf[pl.ds(j * L, L)]
                gidx = ci * CH + j * L + iota
                local = kv - base_cell
                mask = (local >= 0) & (local < cpt) & (gidx < nvalid)
                local = jnp.where(mask, local, 0)
                plsc.store_scatter(mslice, [local], gidx, mask=mask)
                return 0
            lax.fori_loop(0, CH // L, vec, 0)
            return 0
        lax.fori_loop(0, npts_pad // CH, chunk, 0)
        pltpu.async_copy(mslice, map_hbm.at[pl.ds(base_cell, cpt)], sem).wait()

    return functools.partial(
        pl.kernel, body, mesh=_mesh, name=name, compiler_params=_SC_PARAMS,
        out_type=jax.ShapeDtypeStruct((cpt * NW,), jnp.int32),
        scratch_types=[pltpu.VMEM((cpt,), jnp.int32),
                       pltpu.VMEM((CH,), jnp.int32),
                       pltpu.SemaphoreType.DMA])()


def _make_g1(npts, name):
    CH = 128
    per_tile = npts // NW
    nchunks = per_tile // CH
    nk = 27
    KW = 512

    def body(keys_hbm, map_hbm, tbl_hbm, g_hbm,
             kbuf, nkbuf, idxbuf, rb, g512, sem, semw):
        wid = _wid()
        base = wid * per_tile
        zv = jnp.zeros((L,), jnp.float32)

        def zrow(i, _):
            for t in range((KW - nk * C) // L):
                g512[i, pl.ds(nk * C + t * L, L)] = zv
            return 0
        lax.fori_loop(0, CH, zrow, 0)

        def relayout(k, buf):
            def rowcopy(r, _):
                g512[r, pl.ds(k * C, C)] = rb[buf, r, pl.ds(0, C)]
                return 0
            lax.fori_loop(0, CH, rowcopy, 0)

        def chunk(ci, wcnt):
            r0 = base + ci * CH
            pltpu.async_copy(keys_hbm.at[pl.ds(r0, CH)], kbuf, sem).wait()

            def vec(j, _):
                kv = kbuf[pl.ds(j * L, L)]
                for k in range(nk):
                    nkbuf[k, pl.ds(j * L, L)] = kv + DELTAS1[k]
                return 0
            lax.fori_loop(0, CH // L, vec, 0)

            d1 = [pltpu.async_copy(map_hbm.at[nkbuf.at[k]], idxbuf.at[k], sem)
                  for k in range(nk)]
            for d in d1:
                d.wait()
            descs = [None, None]
            descs[0] = pltpu.async_copy(tbl_hbm.at[idxbuf.at[0]], rb.at[0], sem)
            for k in range(nk):
                descs[k % 2].wait()
                if k + 1 < nk:
                    descs[(k + 1) % 2] = pltpu.async_copy(
                        tbl_hbm.at[idxbuf.at[k + 1]], rb.at[(k + 1) % 2], sem)
                relayout(k, k % 2)

            @pl.when(wcnt > 0)
            def _():
                pltpu.make_async_copy(g512, g_hbm.at[pl.ds(0, CH)], semw).wait()
            pltpu.async_copy(g512, g_hbm.at[pl.ds(r0, CH)], semw)
            return wcnt + 1
        nw = lax.fori_loop(0, nchunks, chunk, 0)

        @pl.when(nw > 0)
        def _():
            pltpu.make_async_copy(g512, g_hbm.at[pl.ds(0, CH)], semw).wait()

    return functools.partial(
        pl.kernel, body, mesh=_mesh, name=name, compiler_params=_SC_PARAMS,
        out_type=jax.ShapeDtypeStruct((npts, KW), jnp.float32),
        scratch_types=[pltpu.VMEM((CH,), jnp.int32),
                       pltpu.VMEM((nk, CH), jnp.int32),
                       pltpu.VMEM((nk, CH), jnp.int32),
                       pltpu.VMEM((2, CH, 128), jnp.float32),
                       pltpu.VMEM((CH, KW), jnp.float32),
                       pltpu.SemaphoreType.DMA,
                       pltpu.SemaphoreType.DMA])()


def _make_g2(npts, name):
    CH = 32
    per_tile = npts // NW
    nchunks = per_tile // CH
    nk = 27
    KW = nk * 128

    def body(keys_hbm, map_hbm, tbl_hbm, g_hbm,
             kbuf, nkbuf, idxbuf, gbig, sem, semw):
        wid = _wid()
        base = wid * per_tile

        def chunk(ci, wcnt):
            r0 = base + ci * CH
            pltpu.async_copy(keys_hbm.at[pl.ds(r0, CH)], kbuf, sem).wait()

            def vec(j, _):
                kv = kbuf[pl.ds(j * L, L)]
                for k in range(nk):
                    nkbuf[k, pl.ds(j * L, L)] = kv + DELTAS1[k]
                return 0
            lax.fori_loop(0, CH // L, vec, 0)

            d1 = [pltpu.async_copy(map_hbm.at[nkbuf.at[k]], idxbuf.at[k], sem)
                  for k in range(nk)]
            for d in d1:
                d.wait()

            @pl.when(wcnt > 0)
            def _():
                pltpu.make_async_copy(gbig, g_hbm.at[pl.ds(0, CH)], semw).wait()

            d2 = [pltpu.async_copy(
                tbl_hbm.at[idxbuf.at[k]],
                gbig.at[pl.ds(0, CH), pl.ds(k * 128, 128)], sem)
                  for k in range(nk)]
            for d in d2:
                d.wait()
            pltpu.async_copy(gbig, g_hbm.at[pl.ds(r0, CH)], semw)
            return wcnt + 1
        nw = lax.fori_loop(0, nchunks, chunk, 0)

        @pl.when(nw > 0)
        def _():
            pltpu.make_async_copy(gbig, g_hbm.at[pl.ds(0, CH)], semw).wait()

    return functools.partial(
        pl.kernel, body, mesh=_mesh, name=name, compiler_params=_SC_PARAMS,
        out_type=jax.ShapeDtypeStruct((npts, KW), jnp.float32),
        scratch_types=[pltpu.VMEM((CH,), jnp.int32),
                       pltpu.VMEM((nk, CH), jnp.int32),
                       pltpu.VMEM((nk, CH), jnp.int32),
                       pltpu.VMEM((CH, KW), jnp.float32),
                       pltpu.SemaphoreType.DMA,
                       pltpu.SemaphoreType.DMA])()


def _make_extract(mpad, name):
    CH = 128
    per_tile = mpad // NW
    nchunks = per_tile // CH

    def body(sel_hbm, oc_hbm, out_hbm, sbuf, pbuf, wb, ob, sem):
        wid = _wid()
        base = wid * per_tile
        iota = lax.iota(jnp.int32, L)

        def chunk(ci, _):
            r0 = base + ci * CH
            pltpu.async_copy(sel_hbm.at[pl.ds(r0, CH)], sbuf, sem).wait()

            def vec(j, _):
                sv = sbuf[pl.ds(j * L, L)]
                pbuf[pl.ds(j * L, L)] = lax.shift_right_logical(sv, 3)
                return 0
            lax.fori_loop(0, CH // L, vec, 0)
            pltpu.async_copy(oc_hbm.at[pbuf], wb, sem).wait()

            def vec2(j, _):
                sv = sbuf[pl.ds(j * L, L)]
                rows = j * L + iota
                cbase = (sv & 7) * C
                for c in range(C):
                    v = plsc.load_gather(wb, [rows, cbase + c])
                    plsc.store_scatter(ob, [rows, jnp.full((L,), c, jnp.int32)], v)
                return 0
            lax.fori_loop(0, CH // L, vec2, 0)
            pltpu.async_copy(ob, out_hbm.at[pl.ds(r0, CH)], sem).wait()
            return 0
        lax.fori_loop(0, nchunks, chunk, 0)

    return functools.partial(
        pl.kernel, body, mesh=_mesh, name=name, compiler_params=_SC_PARAMS,
        out_type=jax.ShapeDtypeStruct((mpad, C), jnp.float32),
        scratch_types=[pltpu.VMEM((CH,), jnp.int32),
                       pltpu.VMEM((CH,), jnp.int32),
                       pltpu.VMEM((CH, 128), jnp.float32),
                       pltpu.VMEM((CH, C), jnp.float32),
                       pltpu.SemaphoreType.DMA])()


def _kc_body(g_ref, w_ref, b_ref, o_ref):
    o_ref[...] = jnp.dot(g_ref[...], w_ref[...],
                         preferred_element_type=jnp.float32) + b_ref[...]


def _run_matmul(G, Wr, b, BT, name):
    n, kdim = G.shape
    cout = Wr.shape[1]
    full = lambda s: pl.BlockSpec(s, lambda i: (0,) * len(s))
    return pl.pallas_call(
        _kc_body,
        grid=(n // BT,),
        in_specs=[pl.BlockSpec((BT, kdim), lambda i: (i, 0)),
                  full((kdim, cout)), full((cout,))],
        out_specs=pl.BlockSpec((BT, cout), lambda i: (i, 0)),
        out_shape=jax.ShapeDtypeStruct((n, cout), jnp.float32),
        name=name,
    )(G, Wr, b)


def _kd_body(o1_ref, msk_ref, f_ref):
    o = o1_ref[...]
    msk = msk_ref[...]
    blocks = []
    for j in range(8):
        b = o[:, 16 * j:16 * j + 16]
        m = jnp.mean(b, axis=-1, keepdims=True)
        v = jnp.mean((b - m) ** 2, axis=-1, keepdims=True)
        h = (b - m) * lax.rsqrt(v + 1e-6)
        h = h * jax.nn.sigmoid(h)
        keep = (msk[:, j:j + 1] >= 0).astype(jnp.float32)
        blocks.append(h * keep)
    f_ref[...] = jnp.concatenate(blocks, axis=1)


def _run_kd(out1, selmask):
    n = out1.shape[0]
    BT = 1024
    return pl.pallas_call(
        _kd_body,
        grid=(n // BT,),
        in_specs=[pl.BlockSpec((BT, 128), lambda i: (i, 0)),
                  pl.BlockSpec((BT, 8), lambda i: (i, 0))],
        out_specs=pl.BlockSpec((BT, 128), lambda i: (i, 0)),
        out_shape=jax.ShapeDtypeStruct((n, 128), jnp.float32),
        name="ln2",
    )(out1, selmask)


def _ke_body(g_ref, w_ref, b_ref, x_ref, o_ref):
    o = jnp.dot(g_ref[...], w_ref[...], preferred_element_type=jnp.float32) + b_ref[...]
    x = x_ref[...]
    cols = []
    for a in range(8):
        cols.append(jnp.broadcast_to(x[:, 2 * a:2 * a + 1], (x.shape[0], 8)))
        cols.append(jnp.broadcast_to(x[:, 2 * a + 1:2 * a + 2], (x.shape[0], 8)))
    o_ref[...] = o + jnp.concatenate(cols, axis=1)


def _run_ke(G2, Vall, b2t, x_feats):
    n = G2.shape[0]
    BT = 512
    full = lambda s: pl.BlockSpec(s, lambda i: (0,) * len(s))
    return pl.pallas_call(
        _ke_body,
        grid=(n // BT,),
        in_specs=[pl.BlockSpec((BT, 27 * 128), lambda i: (i, 0)),
                  full((27 * 128, 128)), full((128,)),
                  pl.BlockSpec((BT, 16), lambda i: (i, 0))],
        out_specs=pl.BlockSpec((BT, 128), lambda i: (i, 0)),
        out_shape=jax.ShapeDtypeStruct((n, 128), jnp.float32),
        name="conv2mm",
    )(G2, Vall, b2t, x_feats)


CPT1 = _ceil_to(_ceil_to(T1, NW) // NW, 128)

_KIDX = np.full((27, 8, 8), 27, np.int64)
for _ei, (_ex, _ey, _ez) in enumerate(
        [(x, y, z) for x in (-1, 0, 1) for y in (-1, 0, 1) for z in (-1, 0, 1)]):
    for _b in range(8):
        _bv = ((_b >> 2) & 1, (_b >> 1) & 1, _b & 1)
        for _a in range(8):
            _av = ((_a >> 2) & 1, (_a >> 1) & 1, _a & 1)
            _d = (_bv[0] - _av[0] + 2 * _ex, _bv[1] - _av[1] + 2 * _ey,
                  _bv[2] - _av[2] + 2 * _ez)
            if all(-1 <= t <= 1 for t in _d):
                _KIDX[_ei, _b, _a] = (_d[0] + 1) * 9 + (_d[1] + 1) * 3 + (_d[2] + 1)


def kernel(x_feats, ln1_w, ln1_b, W1, b1, W2, b2, Wsub, bsub, coords, sel_idx):
    N = x_feats.shape[0]
    M = sel_idx.shape[0]
    Mpad = _ceil_to(M, NW * 128)

    subdiv, h1, pk8 = _run_ka(x_feats, coords.astype(jnp.int32), ln1_w, ln1_b,
                              Wsub, bsub)
    pkey = pk8[:, 0]

    sel = sel_idx.astype(jnp.int32)
    selpad = jnp.pad(sel, (0, Mpad - M))
    map1 = _make_map_builder(N, N, CPT1, N, "map1", pad_spread=64)(pkey)
    selmask = _make_map_builder(Mpad, M, (8 * N) // NW, -1, "selmask")(selpad)

    h1w = jnp.pad(h1, ((0, 64), (0, 128 - C)))
    G1 = _make_g1(N, "im2col1")(pkey, map1, h1w)
    W1r = jnp.pad(W1.reshape(27 * C, COUT1), ((0, 512 - 27 * C), (0, 0)))
    out1 = _run_matmul(G1, W1r, b1, 1024, "conv1mm")

    F = _run_kd(out1, selmask.reshape(N, 8))
    Fp = jnp.pad(F, ((0, 64), (0, 0)))
    G2 = _make_g2(N, "im2col2")(pkey, map1, Fp)

    W2z = jnp.concatenate([W2, jnp.zeros((1, C, C), W2.dtype)], axis=0)
    Vall = W2z[jnp.asarray(_KIDX)]
    Vall = Vall.transpose(0, 1, 3, 2, 4).reshape(27 * 128, 128)
    b2t = jnp.tile(b2, 8)

    out_ch = _run_ke(G2, Vall, b2t, x_feats)
    out_chp = jnp.pad(out_ch, ((0, 64), (0, 0)))
    out = _make_extract(Mpad, "extract")(selpad, out_chp)
    return out[:M], subdiv
